# baseline (speedup 1.0000x reference)
import functools

import jax
import jax.numpy as jnp
from jax import lax
from jax.experimental import pallas as pl
from jax.experimental.pallas import tpu as pltpu
from jax.experimental.pallas import tpu_sc as plsc

OP_SHL = 0
OP_SHR = 1
MARK_AX = 2
ALU_LO = 16
ALU_HI = 32
AX_CARRY_LO = 48
OUTPUT_LO = 64
OUTPUT_HI = 80

B = 16384
D = 512
NC = 2
NS = 16
NW = NC * NS
ROWS_PER_W = B // NW
CHUNK = 64
NCHUNKS = ROWS_PER_W // CHUNK
NGROUPS = CHUNK // 16


def _decode_and_patch(buf, lanes, ones):
    for g in range(NGROUPS):
        rows = g * 16 + lanes

        def gathc(col):
            return plsc.load_gather(buf, [rows, jnp.full((16,), col, jnp.int32)])

        def argmax16(start):
            best = gathc(start)
            bidx = jnp.zeros((16,), jnp.int32)
            for j in range(1, 16):
                cur = gathc(start + j)
                m = cur > best
                best = jnp.where(m, cur, best)
                bidx = jnp.where(m, j, bidx)
            return bidx

        c0 = gathc(OP_SHL)
        c1 = gathc(OP_SHR)
        c2 = gathc(MARK_AX)
        a = argmax16(ALU_LO) + 16 * argmax16(ALU_HI)
        sh = jnp.minimum(argmax16(AX_CARRY_LO), 7)
        hi_shr = lax.shift_right_logical(a, sh + 4)

        act_shl = (c0 > 0.5) & (c2 > 0.5)
        act = ((c1 > 0.5) | (c0 > 0.5)) & (c2 > 0.5)
        idx_hi = OUTPUT_HI + jnp.where(act_shl, 0, hi_shr)

        plsc.addupdate_scatter(
            buf, [rows, jnp.full((16,), OUTPUT_LO, jnp.int32)], ones, mask=act)
        plsc.addupdate_scatter(buf, [rows, idx_hi], ones, mask=act)


NBUF = 3


def _sc_body(x_hbm, out_hbm, *scratch):
    bufs = scratch[:NBUF]
    sems_in = scratch[NBUF:2 * NBUF]
    sems_out = scratch[2 * NBUF:]
    wid = lax.axis_index("s") * NC + lax.axis_index("c")
    base = wid * ROWS_PER_W
    lanes = lax.iota(jnp.int32, 16)
    ones = jnp.ones((16,), jnp.float32)

    def start_in(t):
        return pltpu.async_copy(
            x_hbm.at[pl.ds(base + t * CHUNK, CHUNK)], bufs[t % NBUF],
            sems_in[t % NBUF])

    in_flight = {t: start_in(t) for t in range(min(2, NCHUNKS))}
    out_flight = {}

    for t in range(NCHUNKS):
        slot = t % NBUF
        buf = bufs[slot]
        in_flight.pop(t).wait()
        _decode_and_patch(buf, lanes, ones)
        out_flight[t] = pltpu.async_copy(
            buf, out_hbm.at[pl.ds(base + t * CHUNK, CHUNK)], sems_out[slot])
        nxt = t + 2
        if nxt < NCHUNKS:
            prev = nxt - NBUF
            if prev in out_flight:
                out_flight.pop(prev).wait()
            in_flight[nxt] = start_in(nxt)
    for t in sorted(out_flight):
        out_flight[t].wait()


@jax.jit
def kernel(x):
    mesh = plsc.VectorSubcoreMesh(core_axis_name="c", subcore_axis_name="s")
    run = pl.kernel(
        _sc_body,
        out_type=jax.ShapeDtypeStruct((B, D), jnp.float32),
        mesh=mesh,
        scratch_types=(
            [pltpu.VMEM((CHUNK, D), jnp.float32)] * NBUF
            + [pltpu.SemaphoreType.DMA] * (2 * NBUF)
        ),
        compiler_params=pltpu.CompilerParams(
            use_tc_tiling_on_sc=False, needs_layout_passes=False),
    )
    return run(x)


# speedup vs baseline: 3.0508x; 3.0508x over previous
import functools

import jax
import jax.numpy as jnp
from jax import lax
from jax.experimental import pallas as pl

OP_SHL = 0
OP_SHR = 1
MARK_AX = 2
ALU_LO = 16
ALU_HI = 32
AX_CARRY_LO = 48
OUTPUT_LO = 64
OUTPUT_HI = 80

MAGIC32 = 1.5 * float(2 ** 23)


def _magic_floor(x):
    return (x - 0.5 + 0.001) + MAGIC32 - MAGIC32


def _tc_body(x_ref, o_ref):
    xb = x_ref[...]
    br = xb.shape[0]
    li = lax.broadcasted_iota(jnp.int32, (br, 512), 1)

    def argmax16(start):
        mask = (li >= start) & (li < start + 16)
        vals = jnp.where(mask, xb, -1.0)
        m = jnp.max(vals, axis=1, keepdims=True)
        cand = jnp.where(mask & (xb == m), li, 512)
        return jnp.min(cand, axis=1, keepdims=True) - start

    a_lo = argmax16(ALU_LO)
    a_hi = argmax16(ALU_HI)
    sh = argmax16(AX_CARRY_LO)

    op_shl = xb[:, OP_SHL:OP_SHL + 1]
    op_shr = xb[:, OP_SHR:OP_SHR + 1]
    mark_ax = xb[:, MARK_AX:MARK_AX + 1]
    active_shl = (op_shl > 0.5) & (mark_ax > 0.5)
    active_shr = (op_shr > 0.5) & (mark_ax > 0.5)

    a = a_lo + 16 * a_hi
    shv = jnp.minimum(sh, 7)
    hi_shr = lax.shift_right_logical(a, shv + 4)

    idx_lo = jnp.full_like(a, OUTPUT_LO)
    idx_hi = OUTPUT_HI + jnp.where(active_shl, 0, hi_shr)
    active_f = (active_shl | active_shr).astype(jnp.float32)

    delta = jnp.where(li == idx_lo, active_f, 0.0) + jnp.where(
        li == idx_hi, active_f, 0.0)
    o_ref[...] = xb + delta


@functools.partial(jax.jit, static_argnames=("interpret",))
def kernel(x, interpret=False):
    B, D = x.shape
    BR = 2048
    return pl.pallas_call(
        _tc_body,
        grid=(B // BR,),
        in_specs=[pl.BlockSpec((BR, D), lambda i: (i, 0))],
        out_specs=pl.BlockSpec((BR, D), lambda i: (i, 0)),
        out_shape=jax.ShapeDtypeStruct((B, D), x.dtype),
        interpret=interpret,
    )(x)
